# baseline (speedup 1.0000x reference)
import numpy as np
import jax
import jax.numpy as jnp
from jax.experimental import pallas as pl
from jax.experimental.pallas import tpu as pltpu

B, S, H, DIM = 1, 2048, 12, 64
D_STRIDE, L_CMP, L_SLC = 64, 64, 128
WINDOW, BLOCK_Q, TOP_K = 128, 64, 8
N_CMP = (S - L_CMP) // D_STRIDE + 1
N_SLC = (S - L_SLC) // D_STRIDE + 1
QSTEP = 1024
NQS = S // QSTEP
NSEL = QSTEP // BLOCK_Q
WGRP = 128
NWIN = QSTEP // WGRP
WIN_SLAB = WGRP + 2 * WINDOW
WIN_BLKS = WIN_SLAB // D_STRIDE
NEG = float(np.finfo(np.float32).min)
SCALE = DIM ** (-0.5)
BF = jnp.bfloat16


def _slc_map_np():
    alpha, beta = L_SLC // D_STRIDE, L_CMP // D_STRIDE
    M = np.zeros((N_CMP, N_SLC), np.float32)
    for j in range(N_SLC):
        for m in range(alpha):
            for nn in range(beta):
                idx = alpha * j - m - nn
                if 0 <= idx < N_CMP:
                    M[idx, j] += 1.0
    return M


def _softmax(x):
    m = jnp.max(x, axis=-1, keepdims=True)
    e = jnp.exp(x - m)
    return e / jnp.sum(e, axis=-1, keepdims=True)


_DN_NT = (((1,), (1,)), ((), ()))
_DN_NN = (((1,), (0,)), ((), ()))
_PREC = jax.lax.Precision.DEFAULT


def _dot(a, b, dn):
    return jax.lax.dot_general(a.astype(BF), b.astype(BF), dn,
                               precision=_PREC,
                               preferred_element_type=jnp.float32)


def _nsa_body(q_ref, k_ref, v_ref, wkt_ref, wvt_ref, bkv_ref, wg_ref, bg_ref,
              m_ref, o_ref, kc_ref, vc_ref):
    qb = pl.program_id(1)

    @pl.when(qb == 0)
    def _compress():
        row = jax.lax.broadcasted_iota(jnp.int32, (N_CMP, S), 0)
        col = jax.lax.broadcasted_iota(jnp.int32, (N_CMP, S), 1)
        sel = (col // D_STRIDE) == row
        wkm = jnp.where(sel, wkt_ref[...], 0.0)
        wvm = jnp.where(sel, wvt_ref[...], 0.0)
        kc_ref[...] = (_dot(wkm, k_ref[0].reshape(S, DIM), _DN_NN)
                       + bkv_ref[0, 0])
        vc_ref[...] = (_dot(wvm, v_ref[0].reshape(S, DIM), _DN_NN)
                       + bkv_ref[0, 1])

    q_blk = q_ref[0]
    g0 = qb * QSTEP

    kwins, vwins, offs = [], [], []
    for w in range(NWIN):
        wstart = g0 + w * WGRP
        k64 = jnp.clip((wstart - WINDOW) // D_STRIDE, 0,
                       (S - WIN_SLAB) // D_STRIDE)
        offs.append(wstart - k64 * D_STRIDE)
        kwins.append(k_ref[0, pl.ds(k64, WIN_BLKS)].reshape(WIN_SLAB, DIM))
        vwins.append(v_ref[0, pl.ds(k64, WIN_BLKS)].reshape(WIN_SLAB, DIM))

    lc = _dot(q_blk, kc_ref[...], _DN_NT) / SCALE
    ee = (jax.lax.broadcasted_iota(jnp.int32, (WGRP, WIN_SLAB), 1)
          - jax.lax.broadcasted_iota(jnp.int32, (WGRP, WIN_SLAB), 0))
    lws = []
    for w in range(NWIN):
        qw = q_blk[w * WGRP:(w + 1) * WGRP, :]
        lw = _dot(qw, kwins[w], _DN_NT) * SCALE
        lws.append(jnp.where((ee >= offs[w] - WINDOW)
                             & (ee <= offs[w] + WINDOW), lw, NEG))
    gl = _dot(q_blk, wg_ref[...], _DN_NT) + bg_ref[...]

    pc = _softmax(lc)
    pcb = pc.astype(BF)
    pslc = _dot(pcb, m_ref[...], _DN_NN)
    iota_l = jax.lax.broadcasted_iota(jnp.int32, (1, N_SLC), 1)
    rr = jax.lax.broadcasted_iota(jnp.int32, (N_SLC, N_SLC), 0)
    cc = jax.lax.broadcasted_iota(jnp.int32, (N_SLC, N_SLC), 1)
    js_all = []
    for u in range(NSEL):
        scores = jnp.sum(pslc[u * BLOCK_Q:(u + 1) * BLOCK_Q, :],
                         axis=0, keepdims=True)
        st = jnp.transpose(scores)
        above = (st > scores) | ((st == scores) & (rr < cc))
        rank = jnp.sum(above.astype(jnp.float32), axis=0, keepdims=True)
        js_all.append([jnp.min(jnp.where(rank == float(i), iota_l, N_SLC))
                       for i in range(TOP_K)])

    out_cmp = _dot(pcb, vc_ref[...], _DN_NN)

    out_win = jnp.concatenate(
        [_dot(_softmax(lws[w]), vwins[w], _DN_NN) for w in range(NWIN)],
        axis=0)

    lgs_all = [[None] * TOP_K for _ in range(NSEL)]
    for i in range(TOP_K):
        for u in range(NSEL):
            qh = q_blk[u * BLOCK_Q:(u + 1) * BLOCK_Q, :]
            kblk = k_ref[0, pl.ds(js_all[u][i], 2)].reshape(L_SLC, DIM)
            lgs_all[u][i] = _dot(qh, kblk, _DN_NT)
    out_slc_parts = []
    for u in range(NSEL):
        ls = jnp.concatenate(lgs_all[u], axis=1) * SCALE
        psb = _softmax(ls).astype(BF)
        acc = jnp.zeros((BLOCK_Q, DIM), jnp.float32)
        for i in range(TOP_K):
            vblk = v_ref[0, pl.ds(js_all[u][i], 2)].reshape(L_SLC, DIM)
            acc = acc + _dot(psb[:, i * L_SLC:(i + 1) * L_SLC], vblk, _DN_NN)
        out_slc_parts.append(acc)
    out_slc = jnp.concatenate(out_slc_parts, axis=0)

    g = jax.nn.sigmoid(gl)
    o_ref[0] = (g[:, 0:1] * out_cmp + g[:, 1:2] * out_slc
                + g[:, 2:3] * out_win)


def kernel(q, k, v, wk, bk, wv, bv, wg, bg):
    qT = jnp.transpose(q[0].astype(BF), (1, 0, 2))
    kT3 = jnp.transpose(k[0].astype(BF), (1, 0, 2)).reshape(H, N_CMP, L_CMP,
                                                            DIM)
    vT3 = jnp.transpose(v[0].astype(BF), (1, 0, 2)).reshape(H, N_CMP, L_CMP,
                                                            DIM)
    wkt = jnp.tile(wk, N_CMP).reshape(1, S)
    wvt = jnp.tile(wv, N_CMP).reshape(1, S)
    bkv = jnp.stack([bk, bv]).reshape(1, 2)
    bg2 = bg.reshape(1, 3)
    wgb = wg.astype(BF)
    Mmap = jnp.asarray(_slc_map_np()).astype(BF)

    out = pl.pallas_call(
        _nsa_body,
        grid=(H, NQS),
        in_specs=[
            pl.BlockSpec((1, QSTEP, DIM), lambda h, qb: (h, qb, 0)),
            pl.BlockSpec((1, N_CMP, L_CMP, DIM), lambda h, qb: (h, 0, 0, 0)),
            pl.BlockSpec((1, N_CMP, L_CMP, DIM), lambda h, qb: (h, 0, 0, 0)),
            pl.BlockSpec((1, S), lambda h, qb: (0, 0)),
            pl.BlockSpec((1, S), lambda h, qb: (0, 0)),
            pl.BlockSpec((1, 2), lambda h, qb: (0, 0)),
            pl.BlockSpec((3, DIM), lambda h, qb: (0, 0)),
            pl.BlockSpec((1, 3), lambda h, qb: (0, 0)),
            pl.BlockSpec((N_CMP, N_SLC), lambda h, qb: (0, 0)),
        ],
        out_specs=pl.BlockSpec((1, QSTEP, DIM), lambda h, qb: (h, qb, 0)),
        out_shape=jax.ShapeDtypeStruct((H, S, DIM), jnp.float32),
        scratch_shapes=[
            pltpu.VMEM((N_CMP, DIM), jnp.float32),
            pltpu.VMEM((N_CMP, DIM), jnp.float32),
        ],
    )(qT, kT3, vT3, wkt, wvt, bkv, wgb, bg2, Mmap)
    return jnp.transpose(out, (1, 0, 2))[None]


# speedup vs baseline: 1.0281x; 1.0281x over previous
import numpy as np
import jax
import jax.numpy as jnp
from jax.experimental import pallas as pl
from jax.experimental.pallas import tpu as pltpu

B, S, H, DIM = 1, 2048, 12, 64
D_STRIDE, L_CMP, L_SLC = 64, 64, 128
WINDOW, BLOCK_Q, TOP_K = 128, 64, 8
N_CMP = (S - L_CMP) // D_STRIDE + 1
N_SLC = (S - L_SLC) // D_STRIDE + 1
QSTEP = 2048
NQS = S // QSTEP
NSEL = QSTEP // BLOCK_Q
WGRP = 128
NWIN = QSTEP // WGRP
WIN_SLAB = WGRP + 2 * WINDOW
WIN_BLKS = WIN_SLAB // D_STRIDE
NEG = float(np.finfo(np.float32).min)
SCALE = DIM ** (-0.5)
BF = jnp.bfloat16


def _slc_map_np():
    alpha, beta = L_SLC // D_STRIDE, L_CMP // D_STRIDE
    M = np.zeros((N_CMP, N_SLC), np.float32)
    for j in range(N_SLC):
        for m in range(alpha):
            for nn in range(beta):
                idx = alpha * j - m - nn
                if 0 <= idx < N_CMP:
                    M[idx, j] += 1.0
    return M


def _softmax(x):
    m = jnp.max(x, axis=-1, keepdims=True)
    e = jnp.exp(x - m)
    return e / jnp.sum(e, axis=-1, keepdims=True)


_DN_NT = (((1,), (1,)), ((), ()))
_DN_NN = (((1,), (0,)), ((), ()))
_PREC = jax.lax.Precision.DEFAULT


def _dot(a, b, dn):
    return jax.lax.dot_general(a.astype(BF), b.astype(BF), dn,
                               precision=_PREC,
                               preferred_element_type=jnp.float32)


def _nsa_body(q_ref, k_ref, v_ref, wkt_ref, wvt_ref, bkv_ref, wg_ref, bg_ref,
              m_ref, o_ref, kc_ref, vc_ref):
    qb = pl.program_id(1)

    @pl.when(qb == 0)
    def _compress():
        row = jax.lax.broadcasted_iota(jnp.int32, (N_CMP, S), 0)
        col = jax.lax.broadcasted_iota(jnp.int32, (N_CMP, S), 1)
        sel = (col // D_STRIDE) == row
        wkm = jnp.where(sel, wkt_ref[...], 0.0)
        wvm = jnp.where(sel, wvt_ref[...], 0.0)
        kc_ref[...] = (_dot(wkm, k_ref[0].reshape(S, DIM), _DN_NN)
                       + bkv_ref[0, 0])
        vc_ref[...] = (_dot(wvm, v_ref[0].reshape(S, DIM), _DN_NN)
                       + bkv_ref[0, 1])

    q_blk = q_ref[0]
    g0 = qb * QSTEP

    kwins, vwins, offs = [], [], []
    for w in range(NWIN):
        wstart = g0 + w * WGRP
        k64 = jnp.clip((wstart - WINDOW) // D_STRIDE, 0,
                       (S - WIN_SLAB) // D_STRIDE)
        offs.append(wstart - k64 * D_STRIDE)
        kwins.append(k_ref[0, pl.ds(k64, WIN_BLKS)].reshape(WIN_SLAB, DIM))
        vwins.append(v_ref[0, pl.ds(k64, WIN_BLKS)].reshape(WIN_SLAB, DIM))

    lc = _dot(q_blk, kc_ref[...], _DN_NT) / SCALE
    ee = (jax.lax.broadcasted_iota(jnp.int32, (WGRP, WIN_SLAB), 1)
          - jax.lax.broadcasted_iota(jnp.int32, (WGRP, WIN_SLAB), 0))
    lws = []
    for w in range(NWIN):
        qw = q_blk[w * WGRP:(w + 1) * WGRP, :]
        lw = _dot(qw, kwins[w], _DN_NT) * SCALE
        lws.append(jnp.where((ee >= offs[w] - WINDOW)
                             & (ee <= offs[w] + WINDOW), lw, NEG))
    gl = _dot(q_blk, wg_ref[...], _DN_NT) + bg_ref[...]

    pc = _softmax(lc)
    pcb = pc.astype(BF)
    pslc = _dot(pcb, m_ref[...], _DN_NN)
    iota_l = jax.lax.broadcasted_iota(jnp.int32, (1, N_SLC), 1)
    rr = jax.lax.broadcasted_iota(jnp.int32, (N_SLC, N_SLC), 0)
    cc = jax.lax.broadcasted_iota(jnp.int32, (N_SLC, N_SLC), 1)
    js_all = []
    for u in range(NSEL):
        scores = jnp.sum(pslc[u * BLOCK_Q:(u + 1) * BLOCK_Q, :],
                         axis=0, keepdims=True)
        st = jnp.transpose(scores)
        above = (st > scores) | ((st == scores) & (rr < cc))
        rank = jnp.sum(above.astype(jnp.float32), axis=0, keepdims=True)
        js_all.append([jnp.min(jnp.where(rank == float(i), iota_l, N_SLC))
                       for i in range(TOP_K)])

    out_cmp = _dot(pcb, vc_ref[...], _DN_NN)

    out_win = jnp.concatenate(
        [_dot(_softmax(lws[w]), vwins[w], _DN_NN) for w in range(NWIN)],
        axis=0)

    lgs_all = [[None] * TOP_K for _ in range(NSEL)]
    for i in range(TOP_K):
        for u in range(NSEL):
            qh = q_blk[u * BLOCK_Q:(u + 1) * BLOCK_Q, :]
            kblk = k_ref[0, pl.ds(js_all[u][i], 2)].reshape(L_SLC, DIM)
            lgs_all[u][i] = _dot(qh, kblk, _DN_NT)
    out_slc_parts = []
    for u in range(NSEL):
        ls = jnp.concatenate(lgs_all[u], axis=1) * SCALE
        psb = _softmax(ls).astype(BF)
        acc = jnp.zeros((BLOCK_Q, DIM), jnp.float32)
        for i in range(TOP_K):
            vblk = v_ref[0, pl.ds(js_all[u][i], 2)].reshape(L_SLC, DIM)
            acc = acc + _dot(psb[:, i * L_SLC:(i + 1) * L_SLC], vblk, _DN_NN)
        out_slc_parts.append(acc)
    out_slc = jnp.concatenate(out_slc_parts, axis=0)

    g = jax.nn.sigmoid(gl)
    o_ref[0] = (g[:, 0:1] * out_cmp + g[:, 1:2] * out_slc
                + g[:, 2:3] * out_win)


def kernel(q, k, v, wk, bk, wv, bv, wg, bg):
    qT = jnp.transpose(q[0].astype(BF), (1, 0, 2))
    kT3 = jnp.transpose(k[0].astype(BF), (1, 0, 2)).reshape(H, N_CMP, L_CMP,
                                                            DIM)
    vT3 = jnp.transpose(v[0].astype(BF), (1, 0, 2)).reshape(H, N_CMP, L_CMP,
                                                            DIM)
    wkt = jnp.tile(wk, N_CMP).reshape(1, S)
    wvt = jnp.tile(wv, N_CMP).reshape(1, S)
    bkv = jnp.stack([bk, bv]).reshape(1, 2)
    bg2 = bg.reshape(1, 3)
    wgb = wg.astype(BF)
    Mmap = jnp.asarray(_slc_map_np()).astype(BF)

    out = pl.pallas_call(
        _nsa_body,
        grid=(H, NQS),
        in_specs=[
            pl.BlockSpec((1, QSTEP, DIM), lambda h, qb: (h, qb, 0)),
            pl.BlockSpec((1, N_CMP, L_CMP, DIM), lambda h, qb: (h, 0, 0, 0)),
            pl.BlockSpec((1, N_CMP, L_CMP, DIM), lambda h, qb: (h, 0, 0, 0)),
            pl.BlockSpec((1, S), lambda h, qb: (0, 0)),
            pl.BlockSpec((1, S), lambda h, qb: (0, 0)),
            pl.BlockSpec((1, 2), lambda h, qb: (0, 0)),
            pl.BlockSpec((3, DIM), lambda h, qb: (0, 0)),
            pl.BlockSpec((1, 3), lambda h, qb: (0, 0)),
            pl.BlockSpec((N_CMP, N_SLC), lambda h, qb: (0, 0)),
        ],
        out_specs=pl.BlockSpec((1, QSTEP, DIM), lambda h, qb: (h, qb, 0)),
        out_shape=jax.ShapeDtypeStruct((H, S, DIM), jnp.float32),
        scratch_shapes=[
            pltpu.VMEM((N_CMP, DIM), jnp.float32),
            pltpu.VMEM((N_CMP, DIM), jnp.float32),
        ],
    )(qT, kT3, vT3, wkt, wvt, bkv, wgb, bg2, Mmap)
    return jnp.transpose(out, (1, 0, 2))[None]
